# baseline (speedup 1.0000x reference)
import functools

import jax
import jax.numpy as jnp
from jax import lax
from jax.experimental import pallas as pl
from jax.experimental.pallas import tpu as pltpu
from jax.experimental.pallas import tpu_sc as plsc


def _sc_gather(tables, idx_fm, N, D):
    info = plsc.get_sparse_core_info()
    NC, NS = info.num_cores, info.num_subcores
    NW = NC * NS
    W_PER_F = NW // 4
    rows_w = N // W_PER_F
    n_ch = max(1, rows_w // 1024)
    CH = rows_w // n_ch
    mesh = plsc.VectorSubcoreMesh(core_axis_name="c", subcore_axis_name="s")

    @functools.partial(
        pl.kernel,
        mesh=mesh,
        compiler_params=pltpu.CompilerParams(use_tc_tiling_on_sc=False),
        out_type=jax.ShapeDtypeStruct((4 * N, D), jnp.float32),
        scratch_types=[
            pltpu.VMEM((rows_w,), jnp.int32),
            pltpu.VMEM((CH, D), jnp.float32),
            pltpu.SemaphoreType.DMA,
        ],
    )
    def k(t0, t1, t2, t3, idx_hbm, out_hbm, idx_v, rows_v, gsem):
        wid = lax.axis_index("s") * NC + lax.axis_index("c")
        f = wid // W_PER_F
        base = (wid % W_PER_F) * rows_w
        tabs = (t0, t1, t2, t3)
        for ff in range(4):
            @pl.when(f == ff)
            def _():
                pltpu.sync_copy(
                    idx_hbm.at[pl.ds(ff * N + base, rows_w)], idx_v
                )
                for c in range(n_ch):
                    pltpu.async_copy(
                        tabs[ff].at[idx_v.at[pl.ds(c * CH, CH)]], rows_v, gsem
                    ).wait()
                    pltpu.sync_copy(
                        rows_v,
                        out_hbm.at[pl.ds(ff * N + base + c * CH, CH)],
                    )

    return k(*tables, idx_fm)


def _mlp_body(x0_ref, x1_ref, x2_ref, x3_ref, w1_ref, b1_ref, w2_ref, b2_ref,
              *rest, BN, E):
    o_ref = rest[-1]
    xs = [xr[...] for xr in (x0_ref, x1_ref, x2_ref, x3_ref)]
    x_first = jnp.concatenate([xv[:, :E] for xv in xs], axis=1)
    x_second = jnp.concatenate([xv[:, E:] for xv in xs], axis=1)
    x = jnp.concatenate([x_first, x_second], axis=0)
    h = jnp.dot(x, w1_ref[...], preferred_element_type=jnp.float32)
    h = jnp.maximum(h + b1_ref[...], 0.0)
    o_ref[...] = (
        jnp.dot(h, w2_ref[...], preferred_element_type=jnp.float32) + b2_ref[...]
    )


def _tc_mlp(emb2, Nc, E, W1, b1, W2, b2, N, row0_blocks, prev=None):
    H = W1.shape[1]
    O = W2.shape[1]
    BN = 2048
    nb = Nc // BN
    half_blocks = (Nc // 2) // (BN // 2)
    x_specs = [
        pl.BlockSpec(
            (BN // 2, 2 * E),
            functools.partial(lambda i, f: (f * half_blocks + i, 0), f=f),
        )
        for f in range(4)
    ]
    w_specs = [
        pl.BlockSpec((4 * E, H), lambda i: (0, 0)),
        pl.BlockSpec((1, H), lambda i: (0, 0)),
        pl.BlockSpec((H, O), lambda i: (0, 0)),
        pl.BlockSpec((1, O), lambda i: (0, 0)),
    ]
    args = [emb2, emb2, emb2, emb2, W1, b1.reshape(1, H), W2, b2.reshape(1, O)]
    in_specs = x_specs + w_specs
    aliases = {}
    if prev is not None:
        args.append(prev)
        in_specs = in_specs + [pl.BlockSpec(memory_space=pl.ANY)]
        aliases = {8: 0}
    return pl.pallas_call(
        functools.partial(_mlp_body, BN=BN, E=E),
        grid=(nb,),
        in_specs=in_specs,
        out_specs=pl.BlockSpec((BN, O), lambda i: (row0_blocks + i, 0)),
        out_shape=jax.ShapeDtypeStruct((N, O), jnp.float32),
        input_output_aliases=aliases,
    )(*args)


def kernel(input_tensor, genre_table, key_table, meter_table, unl_table, W1, b1, W2, b2):
    N = input_tensor.shape[0]
    V, E = genre_table.shape
    tables = (genre_table, key_table, meter_table, unl_table)
    NCHUNK = 2
    Nc = N // NCHUNK
    BN = 2048
    embs = []
    for c in range(NCHUNK):
        idx_c = (
            input_tensor[c * Nc:(c + 1) * Nc].T
            .reshape(4, Nc // BN, 2, BN // 2)
            .swapaxes(2, 3)
            .reshape(-1)
        )
        embs.append(_sc_gather(tables, idx_c, Nc, E))
    out = None
    for c in range(NCHUNK):
        emb2 = embs[c].reshape(2 * Nc, 2 * E)
        out = _tc_mlp(emb2, Nc, E, W1, b1, W2, b2, N,
                      row0_blocks=c * (Nc // BN), prev=out)
    return out


# speedup vs baseline: 1.0027x; 1.0027x over previous
import functools

import jax
import jax.numpy as jnp
from jax import lax
from jax.experimental import pallas as pl
from jax.experimental.pallas import tpu as pltpu
from jax.experimental.pallas import tpu_sc as plsc


def _sc_gather(tables, idx_fm, N, D):
    info = plsc.get_sparse_core_info()
    NC, NS = info.num_cores, info.num_subcores
    NW = NC * NS
    W_PER_F = NW // 4
    rows_w = N // W_PER_F
    n_ch = max(1, rows_w // 1024)
    CH = rows_w // n_ch
    mesh = plsc.VectorSubcoreMesh(core_axis_name="c", subcore_axis_name="s")

    @functools.partial(
        pl.kernel,
        mesh=mesh,
        compiler_params=pltpu.CompilerParams(use_tc_tiling_on_sc=False),
        out_type=jax.ShapeDtypeStruct((4 * N, D), jnp.float32),
        scratch_types=[
            pltpu.VMEM((rows_w,), jnp.int32),
            pltpu.VMEM((CH, D), jnp.float32),
            pltpu.SemaphoreType.DMA,
        ],
    )
    def k(t0, t1, t2, t3, idx_hbm, out_hbm, idx_v, rows_v, gsem):
        wid = lax.axis_index("s") * NC + lax.axis_index("c")
        f = wid // W_PER_F
        base = (wid % W_PER_F) * rows_w
        tabs = (t0, t1, t2, t3)
        for ff in range(4):
            @pl.when(f == ff)
            def _():
                pltpu.sync_copy(
                    idx_hbm.at[pl.ds(ff * N + base, rows_w)], idx_v
                )
                for c in range(n_ch):
                    pltpu.async_copy(
                        tabs[ff].at[idx_v.at[pl.ds(c * CH, CH)]], rows_v, gsem
                    ).wait()
                    pltpu.sync_copy(
                        rows_v,
                        out_hbm.at[pl.ds(ff * N + base + c * CH, CH)],
                    )

    return k(*tables, idx_fm)


def _mlp_body(x0_ref, x1_ref, x2_ref, x3_ref, w1_ref, b1_ref, w2_ref, b2_ref,
              *rest, BN, E):
    o_ref = rest[-1]
    xs = [xr[...] for xr in (x0_ref, x1_ref, x2_ref, x3_ref)]
    x_first = jnp.concatenate([xv[:, :E] for xv in xs], axis=1)
    x_second = jnp.concatenate([xv[:, E:] for xv in xs], axis=1)
    for half, xh in enumerate((x_first, x_second)):
        h = jnp.dot(xh, w1_ref[...], preferred_element_type=jnp.float32)
        h = jnp.maximum(h + b1_ref[...], 0.0)
        o_ref[pl.ds(half * (BN // 2), BN // 2), :] = (
            jnp.dot(h, w2_ref[...], preferred_element_type=jnp.float32)
            + b2_ref[...]
        )


def _tc_mlp(emb2, Nc, E, W1, b1, W2, b2, N, row0_blocks, prev=None):
    H = W1.shape[1]
    O = W2.shape[1]
    BN = 2048
    nb = Nc // BN
    half_blocks = (Nc // 2) // (BN // 2)
    x_specs = [
        pl.BlockSpec(
            (BN // 2, 2 * E),
            functools.partial(lambda i, f: (f * half_blocks + i, 0), f=f),
        )
        for f in range(4)
    ]
    w_specs = [
        pl.BlockSpec((4 * E, H), lambda i: (0, 0)),
        pl.BlockSpec((1, H), lambda i: (0, 0)),
        pl.BlockSpec((H, O), lambda i: (0, 0)),
        pl.BlockSpec((1, O), lambda i: (0, 0)),
    ]
    args = [emb2, emb2, emb2, emb2, W1, b1.reshape(1, H), W2, b2.reshape(1, O)]
    in_specs = x_specs + w_specs
    aliases = {}
    if prev is not None:
        args.append(prev)
        in_specs = in_specs + [pl.BlockSpec(memory_space=pl.ANY)]
        aliases = {8: 0}
    return pl.pallas_call(
        functools.partial(_mlp_body, BN=BN, E=E),
        grid=(nb,),
        in_specs=in_specs,
        out_specs=pl.BlockSpec((BN, O), lambda i: (row0_blocks + i, 0)),
        out_shape=jax.ShapeDtypeStruct((N, O), jnp.float32),
        input_output_aliases=aliases,
    )(*args)


def kernel(input_tensor, genre_table, key_table, meter_table, unl_table, W1, b1, W2, b2):
    N = input_tensor.shape[0]
    V, E = genre_table.shape
    tables = (genre_table, key_table, meter_table, unl_table)
    NCHUNK = 2
    Nc = N // NCHUNK
    BN = 2048
    embs = []
    for c in range(NCHUNK):
        idx_c = (
            input_tensor[c * Nc:(c + 1) * Nc].T
            .reshape(4, Nc // BN, 2, BN // 2)
            .swapaxes(2, 3)
            .reshape(-1)
        )
        embs.append(_sc_gather(tables, idx_c, Nc, E))
    out = None
    for c in range(NCHUNK):
        emb2 = embs[c].reshape(2 * Nc, 2 * E)
        out = _tc_mlp(emb2, Nc, E, W1, b1, W2, b2, N,
                      row0_blocks=c * (Nc // BN), prev=out)
    return out


# speedup vs baseline: 1.0808x; 1.0779x over previous
import functools

import jax
import jax.numpy as jnp
from jax import lax
from jax.experimental import pallas as pl
from jax.experimental.pallas import tpu as pltpu
from jax.experimental.pallas import tpu_sc as plsc


def _sc_gather(tables, idx_fm, N, D):
    info = plsc.get_sparse_core_info()
    NC, NS = info.num_cores, info.num_subcores
    NW = NC * NS
    W_PER_F = NW // 4
    rows_w = N // W_PER_F
    n_ch = max(1, rows_w // 1024)
    CH = rows_w // n_ch
    mesh = plsc.VectorSubcoreMesh(core_axis_name="c", subcore_axis_name="s")

    @functools.partial(
        pl.kernel,
        mesh=mesh,
        compiler_params=pltpu.CompilerParams(use_tc_tiling_on_sc=False),
        out_type=jax.ShapeDtypeStruct((4 * N, D), jnp.float32),
        scratch_types=[
            pltpu.VMEM((rows_w,), jnp.int32),
            pltpu.VMEM((CH, D), jnp.float32),
            pltpu.SemaphoreType.DMA,
        ],
    )
    def k(t0, t1, t2, t3, idx_hbm, out_hbm, idx_v, rows_v, gsem):
        wid = lax.axis_index("s") * NC + lax.axis_index("c")
        f = wid // W_PER_F
        base = (wid % W_PER_F) * rows_w
        tabs = (t0, t1, t2, t3)
        for ff in range(4):
            @pl.when(f == ff)
            def _():
                pltpu.sync_copy(
                    idx_hbm.at[pl.ds(ff * N + base, rows_w)], idx_v
                )
                for c in range(n_ch):
                    pltpu.async_copy(
                        tabs[ff].at[idx_v.at[pl.ds(c * CH, CH)]], rows_v, gsem
                    ).wait()
                    pltpu.sync_copy(
                        rows_v,
                        out_hbm.at[pl.ds(ff * N + base + c * CH, CH)],
                    )

    return k(*tables, idx_fm)


def _refold(x, BN, E):
    z = jnp.broadcast_to(x[:, None, :], (BN // 2, 2, 2 * E)).reshape(BN, 2 * E)
    even = (lax.broadcasted_iota(jnp.int32, (BN, E), 0) % 2) == 0
    return jnp.where(even, z[:, :E], z[:, E:])


def _mlp_body(x0_ref, x1_ref, x2_ref, x3_ref, w1_ref, b1_ref, w2_ref, b2_ref,
              *rest, BN, E):
    o_ref = rest[-1]
    x = jnp.concatenate(
        [_refold(xr[...], BN, E) for xr in (x0_ref, x1_ref, x2_ref, x3_ref)],
        axis=1,
    )
    h = jnp.dot(x, w1_ref[...], preferred_element_type=jnp.float32)
    h = jnp.maximum(h + b1_ref[...], 0.0)
    o_ref[...] = (
        jnp.dot(h, w2_ref[...], preferred_element_type=jnp.float32) + b2_ref[...]
    )


def _tc_mlp(emb2, Nc, E, W1, b1, W2, b2, N, row0_blocks, prev=None):
    H = W1.shape[1]
    O = W2.shape[1]
    BN = 2048
    nb = Nc // BN
    half_blocks = (Nc // 2) // (BN // 2)
    x_specs = [
        pl.BlockSpec(
            (BN // 2, 2 * E),
            functools.partial(lambda i, f: (f * half_blocks + i, 0), f=f),
        )
        for f in range(4)
    ]
    w_specs = [
        pl.BlockSpec((4 * E, H), lambda i: (0, 0)),
        pl.BlockSpec((1, H), lambda i: (0, 0)),
        pl.BlockSpec((H, O), lambda i: (0, 0)),
        pl.BlockSpec((1, O), lambda i: (0, 0)),
    ]
    args = [emb2, emb2, emb2, emb2, W1, b1.reshape(1, H), W2, b2.reshape(1, O)]
    in_specs = x_specs + w_specs
    aliases = {}
    if prev is not None:
        args.append(prev)
        in_specs = in_specs + [pl.BlockSpec(memory_space=pl.ANY)]
        aliases = {8: 0}
    return pl.pallas_call(
        functools.partial(_mlp_body, BN=BN, E=E),
        grid=(nb,),
        in_specs=in_specs,
        out_specs=pl.BlockSpec((BN, O), lambda i: (row0_blocks + i, 0)),
        out_shape=jax.ShapeDtypeStruct((N, O), jnp.float32),
        input_output_aliases=aliases,
    )(*args)


def kernel(input_tensor, genre_table, key_table, meter_table, unl_table, W1, b1, W2, b2):
    N = input_tensor.shape[0]
    V, E = genre_table.shape
    tables = (genre_table, key_table, meter_table, unl_table)
    NCHUNK = 2
    Nc = N // NCHUNK
    BN = 2048
    embs = []
    for c in range(NCHUNK):
        idx_c = input_tensor[c * Nc:(c + 1) * Nc].T.reshape(-1)
        embs.append(_sc_gather(tables, idx_c, Nc, E))
    out = None
    for c in range(NCHUNK):
        emb2 = embs[c].reshape(2 * Nc, 2 * E)
        out = _tc_mlp(emb2, Nc, E, W1, b1, W2, b2, N,
                      row0_blocks=c * (Nc // BN), prev=out)
    return out


# speedup vs baseline: 1.1425x; 1.0571x over previous
import functools

import jax
import jax.numpy as jnp
from jax import lax
from jax.experimental import pallas as pl
from jax.experimental.pallas import tpu as pltpu
from jax.experimental.pallas import tpu_sc as plsc


def _sc_gather(tables, idx_fm, N, D):
    info = plsc.get_sparse_core_info()
    NC, NS = info.num_cores, info.num_subcores
    NW = NC * NS
    W_PER_F = NW // 4
    rows_w = N // W_PER_F
    n_ch = max(1, rows_w // 1024)
    CH = rows_w // n_ch
    mesh = plsc.VectorSubcoreMesh(core_axis_name="c", subcore_axis_name="s")

    @functools.partial(
        pl.kernel,
        mesh=mesh,
        compiler_params=pltpu.CompilerParams(use_tc_tiling_on_sc=False),
        out_type=jax.ShapeDtypeStruct((4 * N, D), jnp.float32),
        scratch_types=[
            pltpu.VMEM((rows_w,), jnp.int32),
            pltpu.VMEM((CH, D), jnp.float32),
            pltpu.SemaphoreType.DMA,
        ],
    )
    def k(tstk, idx_hbm, out_hbm, idx_v, rows_v, gsem):
        wid = lax.axis_index("s") * NC + lax.axis_index("c")
        f = wid // W_PER_F
        base = (wid % W_PER_F) * rows_w
        for ff in range(4):
            @pl.when(f == ff)
            def _():
                pltpu.sync_copy(
                    idx_hbm.at[pl.ds(ff * N + base, rows_w)], idx_v
                )
                for c in range(n_ch):
                    pltpu.async_copy(
                        tstk.at[ff].at[idx_v.at[pl.ds(c * CH, CH)]],
                        rows_v,
                        gsem,
                    ).wait()
                    pltpu.sync_copy(
                        rows_v,
                        out_hbm.at[pl.ds(ff * N + base + c * CH, CH)],
                    )

    return k(jnp.stack(tables), idx_fm)


def _refold(x, BN, E):
    z = jnp.broadcast_to(x[:, None, :], (BN // 2, 2, 2 * E)).reshape(BN, 2 * E)
    even = (lax.broadcasted_iota(jnp.int32, (BN, E), 0) % 2) == 0
    return jnp.where(even, z[:, :E], z[:, E:])


def _mlp_body(x0_ref, x1_ref, x2_ref, x3_ref, w1_ref, b1_ref, w2_ref, b2_ref,
              *rest, BN, E):
    o_ref = rest[-1]
    x = jnp.concatenate(
        [_refold(xr[...], BN, E) for xr in (x0_ref, x1_ref, x2_ref, x3_ref)],
        axis=1,
    )
    h = jnp.dot(x, w1_ref[...], preferred_element_type=jnp.float32)
    h = jnp.maximum(h + b1_ref[...], 0.0)
    o_ref[...] = (
        jnp.dot(h, w2_ref[...], preferred_element_type=jnp.float32) + b2_ref[...]
    )


def _tc_mlp(emb2, Nc, E, W1, b1, W2, b2, N, row0_blocks, prev=None):
    H = W1.shape[1]
    O = W2.shape[1]
    BN = 2048
    nb = Nc // BN
    half_blocks = (Nc // 2) // (BN // 2)
    x_specs = [
        pl.BlockSpec(
            (BN // 2, 2 * E),
            functools.partial(lambda i, f: (f * half_blocks + i, 0), f=f),
        )
        for f in range(4)
    ]
    w_specs = [
        pl.BlockSpec((4 * E, H), lambda i: (0, 0)),
        pl.BlockSpec((1, H), lambda i: (0, 0)),
        pl.BlockSpec((H, O), lambda i: (0, 0)),
        pl.BlockSpec((1, O), lambda i: (0, 0)),
    ]
    args = [emb2, emb2, emb2, emb2, W1, b1.reshape(1, H), W2, b2.reshape(1, O)]
    in_specs = x_specs + w_specs
    aliases = {}
    if prev is not None:
        args.append(prev)
        in_specs = in_specs + [pl.BlockSpec(memory_space=pl.ANY)]
        aliases = {8: 0}
    return pl.pallas_call(
        functools.partial(_mlp_body, BN=BN, E=E),
        grid=(nb,),
        in_specs=in_specs,
        out_specs=pl.BlockSpec((BN, O), lambda i: (row0_blocks + i, 0)),
        out_shape=jax.ShapeDtypeStruct((N, O), jnp.float32),
        input_output_aliases=aliases,
    )(*args)


def kernel(input_tensor, genre_table, key_table, meter_table, unl_table, W1, b1, W2, b2):
    N = input_tensor.shape[0]
    V, E = genre_table.shape
    tables = (genre_table, key_table, meter_table, unl_table)
    NCHUNK = 2
    Nc = N // NCHUNK
    BN = 2048
    embs = []
    for c in range(NCHUNK):
        idx_c = input_tensor[c * Nc:(c + 1) * Nc].T.reshape(-1)
        embs.append(_sc_gather(tables, idx_c, Nc, E))
    out = None
    for c in range(NCHUNK):
        emb2 = embs[c].reshape(2 * Nc, 2 * E)
        out = _tc_mlp(emb2, Nc, E, W1, b1, W2, b2, N,
                      row0_blocks=c * (Nc // BN), prev=out)
    return out
